# baseline (speedup 1.0000x reference)
import functools
import jax
import jax.numpy as jnp
from jax import lax
from jax.experimental import pallas as pl
from jax.experimental.pallas import tpu as pltpu
from jax.experimental.pallas import tpu_sc as plsc

_NC, _NS, _L = 2, 16, 16
_NW = _NC * _NS
_ROWS, _COLS = 16384, 200
_CPW = _ROWS // _NW
_CH = 128
_K = _CPW // _CH
_NVR = _CH // _L


def _sc_body(x_hbm, fiv_hbm, vs_hbm, out_hbm, buf0, buf1, fiv, vsv,
             in_sem0, in_sem1, out_sem0, out_sem1):
    bufs = (buf0, buf1)
    in_sems = (in_sem0, in_sem1)
    out_sems = (out_sem0, out_sem1)
    wid = lax.axis_index("s") * _NC + lax.axis_index("c")
    base = wid * _CPW
    pltpu.sync_copy(fiv_hbm, fiv)
    pltpu.sync_copy(vs_hbm, vsv)
    fi = fiv[...]
    v = plsc.load_gather(vsv, [fi])
    rv = 1.0 / v.astype(jnp.float32)

    in_h = [None] * _K
    out_h = [None] * _K

    def start_in(k):
        b = k % 2
        in_h[k] = pltpu.make_async_copy(
            x_hbm.at[:, pl.ds(base + k * _CH, _CH)], bufs[b], in_sems[b])
        in_h[k].start()

    def start_out(k):
        b = k % 2
        out_h[k] = pltpu.make_async_copy(
            bufs[b], out_hbm.at[:, pl.ds(base + k * _CH, _CH)], out_sems[b])
        out_h[k].start()

    start_in(0)
    for k in range(_K):
        b = k % 2
        if k + 1 < _K:
            if k >= 1:
                out_h[k - 1].wait()
            start_in(k + 1)
        in_h[k].wait()
        buf = bufs[b]

        @plsc.parallel_loop(0, _COLS, 1, unroll=4)
        def _row(r_i):
            for c in range(0, _CH, _L):
                x = buf[r_i, pl.ds(c, _L)]
                q = (x.astype(jnp.float32) * rv).astype(jnp.int32)
                r = x - q * v
                r = jnp.where(r < 0, r + v, r)
                r = jnp.where(r >= v, r - v, r)
                buf[r_i, pl.ds(c, _L)] = r

        start_out(k)
    out_h[_K - 2].wait()
    out_h[_K - 1].wait()


def kernel(indices, feature_idx, vocab_sizes):
    fiv = jnp.full((_L,), feature_idx, dtype=jnp.int32)
    vsp = vocab_sizes.astype(jnp.int32)
    xt = jnp.swapaxes(indices, 0, 1)
    mesh = plsc.VectorSubcoreMesh(
        core_axis_name="c", subcore_axis_name="s", num_cores=_NC, num_subcores=_NS
    )
    f = functools.partial(
        pl.kernel,
        out_type=jax.ShapeDtypeStruct((_COLS, _ROWS), indices.dtype),
        mesh=mesh,
        compiler_params=pltpu.CompilerParams(use_tc_tiling_on_sc=True, needs_layout_passes=False),
        scratch_types=[
            pltpu.VMEM((_COLS, _CH), jnp.int32),
            pltpu.VMEM((_COLS, _CH), jnp.int32),
            pltpu.VMEM((_L,), jnp.int32),
            pltpu.VMEM((26,), jnp.int32),
            pltpu.SemaphoreType.DMA,
            pltpu.SemaphoreType.DMA,
            pltpu.SemaphoreType.DMA,
            pltpu.SemaphoreType.DMA,
        ],
    )(_sc_body)
    out_t = f(xt, fiv, vsp)
    return jnp.swapaxes(out_t, 0, 1)


# speedup vs baseline: 1.7324x; 1.7324x over previous
import functools
import jax
import jax.numpy as jnp
from jax import lax
from jax.experimental import pallas as pl
from jax.experimental.pallas import tpu as pltpu
from jax.experimental.pallas import tpu_sc as plsc

_NC, _NS, _L = 2, 16, 16
_NW = _NC * _NS
_ROWS, _COLS = 16384, 200
_CPW = _ROWS // _NW
_CH = 128
_K = _CPW // _CH
_NVR = _CH // _L


def _sc_body(x_hbm, fiv_hbm, vs_hbm, out_hbm, buf0, buf1, fiv, vsv,
             in_sem0, in_sem1, out_sem0, out_sem1):
    bufs = (buf0, buf1)
    in_sems = (in_sem0, in_sem1)
    out_sems = (out_sem0, out_sem1)
    wid = lax.axis_index("s") * _NC + lax.axis_index("c")
    base = wid * _CPW
    pltpu.sync_copy(fiv_hbm, fiv)
    pltpu.sync_copy(vs_hbm, vsv)
    fi = fiv[...]
    v = plsc.load_gather(vsv, [fi])
    rv = 1.0 / v.astype(jnp.float32)

    in_h = [None] * _K
    out_h = [None] * _K

    def start_in(k):
        b = k % 2
        in_h[k] = pltpu.make_async_copy(
            x_hbm.at[:, pl.ds(base + k * _CH, _CH)], bufs[b], in_sems[b])
        in_h[k].start()

    def start_out(k):
        b = k % 2
        out_h[k] = pltpu.make_async_copy(
            bufs[b], out_hbm.at[:, pl.ds(base + k * _CH, _CH)], out_sems[b])
        out_h[k].start()

    start_in(0)
    for k in range(_K):
        b = k % 2
        if k + 1 < _K:
            if k >= 1:
                out_h[k - 1].wait()
            start_in(k + 1)
        in_h[k].wait()
        buf = bufs[b]

        @plsc.parallel_loop(0, _COLS, 1, unroll=2)
        def _row(r_i):
            for c in range(0, _CH, _L):
                x = buf[r_i, pl.ds(c, _L)]
                q = (x.astype(jnp.float32) * rv).astype(jnp.int32)
                r = x - q * v
                r = jnp.where(r < 0, r + v, r)
                r = jnp.where(r >= v, r - v, r)
                buf[r_i, pl.ds(c, _L)] = r

        start_out(k)
    out_h[_K - 2].wait()
    out_h[_K - 1].wait()


def kernel(indices, feature_idx, vocab_sizes):
    fiv = jnp.full((_L,), feature_idx, dtype=jnp.int32)
    vsp = vocab_sizes.astype(jnp.int32)
    xt = jnp.swapaxes(indices, 0, 1)
    mesh = plsc.VectorSubcoreMesh(
        core_axis_name="c", subcore_axis_name="s", num_cores=_NC, num_subcores=_NS
    )
    f = functools.partial(
        pl.kernel,
        out_type=jax.ShapeDtypeStruct((_COLS, _ROWS), indices.dtype),
        mesh=mesh,
        compiler_params=pltpu.CompilerParams(use_tc_tiling_on_sc=True, needs_layout_passes=False),
        scratch_types=[
            pltpu.VMEM((_COLS, _CH), jnp.int32),
            pltpu.VMEM((_COLS, _CH), jnp.int32),
            pltpu.VMEM((_L,), jnp.int32),
            pltpu.VMEM((26,), jnp.int32),
            pltpu.SemaphoreType.DMA,
            pltpu.SemaphoreType.DMA,
            pltpu.SemaphoreType.DMA,
            pltpu.SemaphoreType.DMA,
        ],
    )(_sc_body)
    out_t = f(xt, fiv, vsp)
    return jnp.swapaxes(out_t, 0, 1)


# speedup vs baseline: 1.8512x; 1.0686x over previous
import functools
import jax
import jax.numpy as jnp
from jax import lax
from jax.experimental import pallas as pl
from jax.experimental.pallas import tpu as pltpu
from jax.experimental.pallas import tpu_sc as plsc

_NC, _NS, _L = 2, 16, 16
_NW = _NC * _NS
_ROWS, _COLS = 16384, 200
_CPW = _ROWS // _NW
_CH = 128
_K = _CPW // _CH
_NVR = _CH // _L


def _sc_body(x_hbm, fiv_hbm, vs_hbm, out_hbm, buf0, buf1, fiv, vsv,
             in_sem0, in_sem1, out_sem0, out_sem1):
    bufs = (buf0, buf1)
    in_sems = (in_sem0, in_sem1)
    out_sems = (out_sem0, out_sem1)
    wid = lax.axis_index("s") * _NC + lax.axis_index("c")
    base = wid * _CPW
    pltpu.sync_copy(fiv_hbm, fiv)
    pltpu.sync_copy(vs_hbm, vsv)
    fi = fiv[...]
    v = plsc.load_gather(vsv, [fi])
    rv = (1.0 / v.astype(jnp.float32)) * (1.0 - 2.0 ** -21)
    vu = plsc.bitcast(v, jnp.uint32)

    in_h = [None] * _K
    out_h = [None] * _K

    def start_in(k):
        b = k % 2
        in_h[k] = pltpu.make_async_copy(
            x_hbm.at[:, pl.ds(base + k * _CH, _CH)], bufs[b], in_sems[b])
        in_h[k].start()

    def start_out(k):
        b = k % 2
        out_h[k] = pltpu.make_async_copy(
            bufs[b], out_hbm.at[:, pl.ds(base + k * _CH, _CH)], out_sems[b])
        out_h[k].start()

    start_in(0)
    for k in range(_K):
        b = k % 2
        if k + 1 < _K:
            if k >= 1:
                out_h[k - 1].wait()
            start_in(k + 1)
        in_h[k].wait()
        buf = bufs[b]

        @plsc.parallel_loop(0, _COLS, 1, unroll=2)
        def _row(r_i):
            for c in range(0, _CH, _L):
                x = buf[r_i, pl.ds(c, _L)]
                q = (x.astype(jnp.float32) * rv).astype(jnp.int32)
                ru = plsc.bitcast(x - q * v, jnp.uint32)
                ru = jnp.minimum(ru, ru - vu)
                ru = jnp.minimum(ru, ru - vu)
                buf[r_i, pl.ds(c, _L)] = plsc.bitcast(ru, jnp.int32)

        start_out(k)
    out_h[_K - 2].wait()
    out_h[_K - 1].wait()


def kernel(indices, feature_idx, vocab_sizes):
    fiv = jnp.full((_L,), feature_idx, dtype=jnp.int32)
    vsp = vocab_sizes.astype(jnp.int32)
    xt = jnp.swapaxes(indices, 0, 1)
    mesh = plsc.VectorSubcoreMesh(
        core_axis_name="c", subcore_axis_name="s", num_cores=_NC, num_subcores=_NS
    )
    f = functools.partial(
        pl.kernel,
        out_type=jax.ShapeDtypeStruct((_COLS, _ROWS), indices.dtype),
        mesh=mesh,
        compiler_params=pltpu.CompilerParams(use_tc_tiling_on_sc=True, needs_layout_passes=False),
        scratch_types=[
            pltpu.VMEM((_COLS, _CH), jnp.int32),
            pltpu.VMEM((_COLS, _CH), jnp.int32),
            pltpu.VMEM((_L,), jnp.int32),
            pltpu.VMEM((26,), jnp.int32),
            pltpu.SemaphoreType.DMA,
            pltpu.SemaphoreType.DMA,
            pltpu.SemaphoreType.DMA,
            pltpu.SemaphoreType.DMA,
        ],
    )(_sc_body)
    out_t = f(xt, fiv, vsp)
    return jnp.swapaxes(out_t, 0, 1)
